# Initial kernel scaffold; baseline (speedup 1.0000x reference)
import jax
import jax.numpy as jnp
from jax.experimental import pallas as pl


def kernel(token_ids, time_deltas, patient_contexts, token_table, t2v_lin_w, t2v_lin_b, t2v_freq_w, t2v_freq_b, time_proj_w, ctx_token, context_proj_w, ln_gamma, ln_beta):
    raise NotImplementedError("write your pallas kernel here")


# speedup vs baseline: 1.6255x; 1.6255x over previous
import functools
import math

import jax
import jax.numpy as jnp
from jax import lax
from jax.experimental import pallas as pl
from jax.experimental.pallas import tpu as pltpu
from jax.experimental.pallas import tpu_sc as plsc

_VOCAB = 100000
_CTX_DIM = 32
_T2V_DIM = 8
_D = 128
_B = 1024
_T = 200


def _sc_gather(table, idx_flat):
    info = plsc.get_sparse_core_info()
    nc, ns = info.num_cores, info.num_subcores
    nw = nc * ns
    n = idx_flat.shape[0]
    d = table.shape[1]
    n_per_w = n // nw
    ch = 320
    n_ch = n_per_w // ch
    mesh = plsc.VectorSubcoreMesh(core_axis_name="c", subcore_axis_name="s")

    @functools.partial(
        pl.kernel,
        mesh=mesh,
        out_type=jax.ShapeDtypeStruct((n, d), jnp.float32),
        scratch_types=[
            pltpu.VMEM((ch,), jnp.int32),
            pltpu.VMEM((ch, d), jnp.float32),
            pltpu.SemaphoreType.DMA,
        ],
    )
    def k(table_hbm, idx_hbm, out_hbm, idx_v, rows_v, sem):
        wid = lax.axis_index("s") * nc + lax.axis_index("c")
        base = wid * n_per_w

        def body(i, carry):
            off = base + i * ch
            pltpu.sync_copy(idx_hbm.at[pl.ds(off, ch)], idx_v)
            pltpu.async_copy(table_hbm.at[idx_v], rows_v, sem).wait()
            pltpu.sync_copy(rows_v, out_hbm.at[pl.ds(off, ch)])
            return carry

        lax.fori_loop(0, n_ch, body, 0)

    return k(table, idx_flat)


def _dense_body(tok_ref, td_ref, pc_ref, wt_ref, cw_ref, ctxtok_ref,
                lw_ref, lb_ref, fw_ref, fb_ref, gamma_ref, beta_ref, out_ref):
    bb = pc_ref.shape[0]
    scale_inv = 1.0 / math.sqrt(_D)
    eps = 1e-5
    gamma = gamma_ref[:].reshape(1, _D)
    beta = beta_ref[:].reshape(1, _D)

    def layernorm(x):
        mean = jnp.mean(x, axis=-1, keepdims=True)
        xc = x - mean
        var = jnp.mean(xc * xc, axis=-1, keepdims=True)
        return xc * lax.rsqrt(var + eps) * gamma + beta

    ctx = ctxtok_ref[:].reshape(1, _D) + jnp.dot(
        pc_ref[:], cw_ref[:], preferred_element_type=jnp.float32)
    ctx_n = layernorm(ctx)

    t = td_ref[:]
    lin = t * lw_ref[0, 0] + lb_ref[0]
    per = jnp.sin(t * fw_ref[0:1, :] + fb_ref[0:1, :])
    time_vec = lin * wt_ref[0:1, :] + jnp.dot(
        per, wt_ref[1:, :], preferred_element_type=jnp.float32)
    ev = (tok_ref[:] + time_vec) * scale_inv
    ev_n = layernorm(ev)

    out_ref[:, 0, :] = ctx_n
    for j in range(bb):
        out_ref[j, 1:, :] = ev_n[j * _T:(j + 1) * _T, :]


def _dense(tok_flat, td2, patient_contexts, wt, cw, ctx_token,
           lw, lb, fw, fb, gamma, beta):
    bb = 8
    grid = (_B // bb,)
    return pl.pallas_call(
        _dense_body,
        grid=grid,
        in_specs=[
            pl.BlockSpec((bb * _T, _D), lambda i: (i, 0)),
            pl.BlockSpec((bb * _T, 1), lambda i: (i, 0)),
            pl.BlockSpec((bb, _CTX_DIM), lambda i: (i, 0)),
            pl.BlockSpec((_T2V_DIM, _D), lambda i: (0, 0)),
            pl.BlockSpec((_CTX_DIM, _D), lambda i: (0, 0)),
            pl.BlockSpec((_D,), lambda i: (0,)),
            pl.BlockSpec((1, 1), lambda i: (0, 0)),
            pl.BlockSpec((1,), lambda i: (0,)),
            pl.BlockSpec((1, _T2V_DIM - 1), lambda i: (0, 0)),
            pl.BlockSpec((1, _T2V_DIM - 1), lambda i: (0, 0)),
            pl.BlockSpec((_D,), lambda i: (0,)),
            pl.BlockSpec((_D,), lambda i: (0,)),
        ],
        out_specs=pl.BlockSpec((bb, _T + 1, _D), lambda i: (i, 0, 0)),
        out_shape=jax.ShapeDtypeStruct((_B, _T + 1, _D), jnp.float32),
    )(tok_flat, td2, patient_contexts, wt, cw, ctx_token,
      lw, lb, fw, fb, gamma, beta)


def kernel(token_ids, time_deltas, patient_contexts, token_table, t2v_lin_w,
           t2v_lin_b, t2v_freq_w, t2v_freq_b, time_proj_w, ctx_token,
           context_proj_w, ln_gamma, ln_beta):
    idx_flat = token_ids.reshape(-1)
    tok_flat = _sc_gather(token_table, idx_flat)
    td2 = time_deltas.reshape(-1, 1)
    wt = time_proj_w.T
    cw = context_proj_w.T
    fw = t2v_freq_w.reshape(1, -1)
    fb = t2v_freq_b.reshape(1, -1)
    return _dense(tok_flat, td2, patient_contexts, wt, cw, ctx_token,
                  t2v_lin_w, t2v_lin_b, fw, fb, ln_gamma, ln_beta)


# speedup vs baseline: 2.9969x; 1.8437x over previous
import functools
import math

import jax
import jax.numpy as jnp
from jax import lax
from jax.experimental import pallas as pl
from jax.experimental.pallas import tpu as pltpu
from jax.experimental.pallas import tpu_sc as plsc

_VOCAB = 100000
_CTX_DIM = 32
_T2V_DIM = 8
_D = 128
_B = 1024
_T = 200


def _sc_gather(table, idx_flat):
    info = plsc.get_sparse_core_info()
    nc, ns = info.num_cores, info.num_subcores
    nw = nc * ns
    n = idx_flat.shape[0]
    d = table.shape[1]
    n_per_w = n // nw
    ch = 320
    n_ch = n_per_w // ch
    mesh = plsc.VectorSubcoreMesh(core_axis_name="c", subcore_axis_name="s")

    @functools.partial(
        pl.kernel,
        mesh=mesh,
        out_type=jax.ShapeDtypeStruct((n, d), jnp.float32),
        scratch_types=[
            pltpu.VMEM((ch,), jnp.int32),
            pltpu.VMEM((ch, d), jnp.float32),
            pltpu.SemaphoreType.DMA,
        ],
    )
    def k(table_hbm, idx_hbm, out_hbm, idx_v, rows_v, sem):
        wid = lax.axis_index("s") * nc + lax.axis_index("c")
        base = wid * n_per_w

        def body(i, carry):
            off = base + i * ch
            pltpu.sync_copy(idx_hbm.at[pl.ds(off, ch)], idx_v)
            pltpu.async_copy(table_hbm.at[idx_v], rows_v, sem).wait()
            pltpu.sync_copy(rows_v, out_hbm.at[pl.ds(off, ch)])
            return carry

        lax.fori_loop(0, n_ch, body, 0)

    return k(table, idx_flat)


def _dense_body(tok_ref, td_ref, pc_ref, wt_ref, cw_ref, ctxtok_ref,
                fw_ref, fb_ref, gamma_ref, beta_ref, out_ref):
    bb = pc_ref.shape[0]
    scale_inv = 1.0 / math.sqrt(_D)
    eps = 1e-5
    gamma = gamma_ref[:].reshape(1, _D)
    beta = beta_ref[:].reshape(1, _D)

    def layernorm(x):
        mean = jnp.mean(x, axis=-1, keepdims=True)
        xc = x - mean
        var = jnp.mean(xc * xc, axis=-1, keepdims=True)
        return xc * lax.rsqrt(var + eps) * gamma + beta

    ctx = ctxtok_ref[:].reshape(1, _D) + jnp.dot(
        pc_ref[:], cw_ref[:], preferred_element_type=jnp.float32)
    ctx_n = layernorm(ctx)

    n = td_ref.shape[2]
    args8 = jnp.broadcast_to(td_ref[0], (_T2V_DIM, n)) * fw_ref[:] + fb_ref[:]
    s8 = jnp.sin(args8)
    rowmask = lax.broadcasted_iota(jnp.int32, (_T2V_DIM, n), 0) == 0
    pt = jnp.where(rowmask, args8, s8)
    time_vec = lax.dot_general(
        pt, wt_ref[:], (((0,), (0,)), ((), ())),
        preferred_element_type=jnp.float32)
    ev = (tok_ref[:] + time_vec) * scale_inv
    ev_n = layernorm(ev)

    out_ref[:, 0, :] = ctx_n
    for j in range(bb):
        out_ref[j, 1:, :] = ev_n[j * _T:(j + 1) * _T, :]


def _dense(tok_flat, td2, patient_contexts, wt, cw, ctx_token,
           fw, fb, gamma, beta):
    bb = 8
    grid = (_B // bb,)
    return pl.pallas_call(
        _dense_body,
        grid=grid,
        in_specs=[
            pl.BlockSpec((bb * _T, _D), lambda i: (i, 0)),
            pl.BlockSpec((1, 1, bb * _T), lambda i: (i, 0, 0)),
            pl.BlockSpec((bb, _CTX_DIM), lambda i: (i, 0)),
            pl.BlockSpec((_T2V_DIM, _D), lambda i: (0, 0)),
            pl.BlockSpec((_CTX_DIM, _D), lambda i: (0, 0)),
            pl.BlockSpec((_D,), lambda i: (0,)),
            pl.BlockSpec((_T2V_DIM, 1), lambda i: (0, 0)),
            pl.BlockSpec((_T2V_DIM, 1), lambda i: (0, 0)),
            pl.BlockSpec((_D,), lambda i: (0,)),
            pl.BlockSpec((_D,), lambda i: (0,)),
        ],
        out_specs=pl.BlockSpec((bb, _T + 1, _D), lambda i: (i, 0, 0)),
        out_shape=jax.ShapeDtypeStruct((_B, _T + 1, _D), jnp.float32),
    )(tok_flat, td2, patient_contexts, wt, cw, ctx_token,
      fw, fb, gamma, beta)


def kernel(token_ids, time_deltas, patient_contexts, token_table, t2v_lin_w,
           t2v_lin_b, t2v_freq_w, t2v_freq_b, time_proj_w, ctx_token,
           context_proj_w, ln_gamma, ln_beta):
    idx_flat = token_ids.reshape(-1)
    tok_flat = _sc_gather(token_table, idx_flat)
    td2 = time_deltas.reshape(_B // 8, 1, 8 * _T)
    wt = time_proj_w.T
    cw = context_proj_w.T
    fw = jnp.concatenate([t2v_lin_w.reshape(1), t2v_freq_w.reshape(-1)],
                         axis=0).reshape(_T2V_DIM, 1)
    fb = jnp.concatenate([t2v_lin_b, t2v_freq_b], axis=0).reshape(_T2V_DIM, 1)
    return _dense(tok_flat, td2, patient_contexts, wt, cw, ctx_token,
                  fw, fb, ln_gamma, ln_beta)


# speedup vs baseline: 3.5224x; 1.1753x over previous
import functools
import math

import jax
import jax.numpy as jnp
from jax import lax
from jax.experimental import pallas as pl
from jax.experimental.pallas import tpu as pltpu
from jax.experimental.pallas import tpu_sc as plsc

_VOCAB = 100000
_CTX_DIM = 32
_T2V_DIM = 8
_D = 128
_B = 1024
_T = 200


def _sc_gather(table, idx_flat):
    info = plsc.get_sparse_core_info()
    nc, ns = info.num_cores, info.num_subcores
    nw = nc * ns
    n = idx_flat.shape[0]
    d = table.shape[1]
    n_per_w = n // nw
    ch = 320
    n_ch = n_per_w // ch
    mesh = plsc.VectorSubcoreMesh(core_axis_name="c", subcore_axis_name="s")

    @functools.partial(
        pl.kernel,
        mesh=mesh,
        out_type=jax.ShapeDtypeStruct((n, d), jnp.float32),
        scratch_types=[
            pltpu.VMEM((ch,), jnp.int32),
            pltpu.VMEM((ch, d), jnp.float32),
            pltpu.SemaphoreType.DMA,
        ],
    )
    def k(table_hbm, idx_hbm, out_hbm, idx_v, rows_v, sem):
        wid = lax.axis_index("s") * nc + lax.axis_index("c")
        base = wid * n_per_w

        def body(i, carry):
            off = base + i * ch
            pltpu.sync_copy(idx_hbm.at[pl.ds(off, ch)], idx_v)
            pltpu.async_copy(table_hbm.at[idx_v], rows_v, sem).wait()
            pltpu.sync_copy(rows_v, out_hbm.at[pl.ds(off, ch)])
            return carry

        lax.fori_loop(0, n_ch, body, 0)

    return k(table, idx_flat)


def _dense_body(tok_ref, td_ref, pc_ref, wt_ref, cw_ref, ctxtok_ref,
                fw_ref, fb_ref, gamma_ref, beta_ref, out_ref):
    bb = pc_ref.shape[0]
    scale_inv = 1.0 / math.sqrt(_D)
    eps = 1e-5
    gamma = gamma_ref[:].reshape(1, _D)
    beta = beta_ref[:].reshape(1, _D)

    ctx = ctxtok_ref[:].reshape(1, _D) + jnp.dot(
        pc_ref[:], cw_ref[:], preferred_element_type=jnp.float32)

    n = td_ref.shape[2]
    args8 = jnp.broadcast_to(td_ref[0], (_T2V_DIM, n)) * fw_ref[:] + fb_ref[:]
    s8 = jnp.sin(args8)
    rowmask = lax.broadcasted_iota(jnp.int32, (_T2V_DIM, n), 0) == 0
    pt = jnp.where(rowmask, args8, s8)
    time_vec = lax.dot_general(
        pt, wt_ref[:], (((0,), (0,)), ((), ())),
        preferred_element_type=jnp.float32)
    ev = (tok_ref[:] + time_vec) * scale_inv

    x = jnp.concatenate([ctx, ev], axis=0)
    jmat = jnp.full((_D, _D), 1.0 / _D, dtype=jnp.float32)
    m1 = jnp.dot(x, jmat, preferred_element_type=jnp.float32)
    m2 = jnp.dot(x * x, jmat, preferred_element_type=jnp.float32)
    var = m2 - m1 * m1
    x_n = (x - m1) * lax.rsqrt(var + eps) * gamma + beta

    out_ref[:, 0, :] = x_n[:bb, :]
    for j in range(bb):
        out_ref[j, 1:, :] = x_n[bb + j * _T:bb + (j + 1) * _T, :]


def _dense(tok_flat, td2, patient_contexts, wt, cw, ctx_token,
           fw, fb, gamma, beta):
    bb = 16
    grid = (_B // bb,)
    return pl.pallas_call(
        _dense_body,
        grid=grid,
        in_specs=[
            pl.BlockSpec((bb * _T, _D), lambda i: (i, 0)),
            pl.BlockSpec((1, 1, bb * _T), lambda i: (i, 0, 0)),
            pl.BlockSpec((bb, _CTX_DIM), lambda i: (i, 0)),
            pl.BlockSpec((_T2V_DIM, _D), lambda i: (0, 0)),
            pl.BlockSpec((_CTX_DIM, _D), lambda i: (0, 0)),
            pl.BlockSpec((_D,), lambda i: (0,)),
            pl.BlockSpec((_T2V_DIM, 1), lambda i: (0, 0)),
            pl.BlockSpec((_T2V_DIM, 1), lambda i: (0, 0)),
            pl.BlockSpec((_D,), lambda i: (0,)),
            pl.BlockSpec((_D,), lambda i: (0,)),
        ],
        out_specs=pl.BlockSpec((bb, _T + 1, _D), lambda i: (i, 0, 0)),
        out_shape=jax.ShapeDtypeStruct((_B, _T + 1, _D), jnp.float32),
    )(tok_flat, td2, patient_contexts, wt, cw, ctx_token,
      fw, fb, gamma, beta)


def kernel(token_ids, time_deltas, patient_contexts, token_table, t2v_lin_w,
           t2v_lin_b, t2v_freq_w, t2v_freq_b, time_proj_w, ctx_token,
           context_proj_w, ln_gamma, ln_beta):
    idx_flat = token_ids.reshape(-1)
    tok_flat = _sc_gather(token_table, idx_flat)
    td2 = time_deltas.reshape(_B // 16, 1, 16 * _T)
    wt = time_proj_w.T
    cw = context_proj_w.T
    fw = jnp.concatenate([t2v_lin_w.reshape(1), t2v_freq_w.reshape(-1)],
                         axis=0).reshape(_T2V_DIM, 1)
    fb = jnp.concatenate([t2v_lin_b, t2v_freq_b], axis=0).reshape(_T2V_DIM, 1)
    return _dense(tok_flat, td2, patient_contexts, wt, cw, ctx_token,
                  fw, fb, ln_gamma, ln_beta)


# speedup vs baseline: 3.8110x; 1.0819x over previous
import functools
import math

import jax
import jax.numpy as jnp
from jax import lax
from jax.experimental import pallas as pl
from jax.experimental.pallas import tpu as pltpu
from jax.experimental.pallas import tpu_sc as plsc

_VOCAB = 100000
_CTX_DIM = 32
_T2V_DIM = 8
_D = 128
_B = 1024
_T = 200


def _sc_gather(table, idx_flat):
    info = plsc.get_sparse_core_info()
    nc, ns = info.num_cores, info.num_subcores
    nw = nc * ns
    n = idx_flat.shape[0]
    d = table.shape[1]
    n_per_w = n // nw
    ch = 320
    n_ch = n_per_w // ch
    mesh = plsc.VectorSubcoreMesh(core_axis_name="c", subcore_axis_name="s")

    @functools.partial(
        pl.kernel,
        mesh=mesh,
        out_type=jax.ShapeDtypeStruct((n, d), jnp.float32),
        scratch_types=[
            pltpu.VMEM((ch,), jnp.int32),
            pltpu.VMEM((ch, d), jnp.float32),
            pltpu.SemaphoreType.DMA,
        ],
    )
    def k(table_hbm, idx_hbm, out_hbm, idx_v, rows_v, sem):
        wid = lax.axis_index("s") * nc + lax.axis_index("c")
        base = wid * n_per_w

        def body(i, carry):
            off = base + i * ch
            pltpu.sync_copy(idx_hbm.at[pl.ds(off, ch)], idx_v)
            pltpu.async_copy(table_hbm.at[idx_v], rows_v, sem).wait()
            pltpu.sync_copy(rows_v, out_hbm.at[pl.ds(off, ch)])
            return carry

        lax.fori_loop(0, n_ch, body, 0)

    return k(table, idx_flat)


def _dense_body(tok_ref, td_ref, pc_ref, wt_ref, cw_ref, ctxtok_ref,
                fw_ref, fb_ref, gamma_ref, beta_ref, out_ref):
    bb = pc_ref.shape[0]
    scale_inv = 1.0 / math.sqrt(_D)
    eps = 1e-5
    gamma = gamma_ref[:].reshape(1, _D)
    beta = beta_ref[:].reshape(1, _D)

    ctx = ctxtok_ref[:].reshape(1, _D) + jnp.dot(
        pc_ref[:], cw_ref[:], preferred_element_type=jnp.float32)

    n = td_ref.shape[2]
    args8 = jnp.broadcast_to(td_ref[0], (_T2V_DIM, n)) * fw_ref[:] + fb_ref[:]
    s8 = jnp.sin(args8)
    rowmask = lax.broadcasted_iota(jnp.int32, (_T2V_DIM, n), 0) == 0
    pt = jnp.where(rowmask, args8, s8)
    time_vec = lax.dot_general(
        pt, wt_ref[:], (((0,), (0,)), ((), ())),
        preferred_element_type=jnp.float32)
    ev = (tok_ref[:] + time_vec) * scale_inv

    x = jnp.concatenate([ctx, ev], axis=0)
    jmat = jnp.full((_D, _D), 1.0 / _D, dtype=jnp.float32)
    m1 = jnp.dot(x, jmat, preferred_element_type=jnp.float32)
    m2 = jnp.dot(x * x, jmat, preferred_element_type=jnp.float32)
    var = m2 - m1 * m1
    x_n = (x - m1) * lax.rsqrt(var + eps) * gamma + beta

    out_ref[:, 0, :] = x_n[:bb, :]
    for j in range(bb):
        out_ref[j, 1:, :] = x_n[bb + j * _T:bb + (j + 1) * _T, :]


def _dense(tok_flat, td2, patient_contexts, wt, cw, ctx_token,
           fw, fb, gamma, beta):
    bb = 32
    grid = (_B // bb,)
    return pl.pallas_call(
        _dense_body,
        grid=grid,
        in_specs=[
            pl.BlockSpec((bb * _T, _D), lambda i: (i, 0)),
            pl.BlockSpec((1, 1, bb * _T), lambda i: (i, 0, 0)),
            pl.BlockSpec((bb, _CTX_DIM), lambda i: (i, 0)),
            pl.BlockSpec((_T2V_DIM, _D), lambda i: (0, 0)),
            pl.BlockSpec((_CTX_DIM, _D), lambda i: (0, 0)),
            pl.BlockSpec((_D,), lambda i: (0,)),
            pl.BlockSpec((_T2V_DIM, 1), lambda i: (0, 0)),
            pl.BlockSpec((_T2V_DIM, 1), lambda i: (0, 0)),
            pl.BlockSpec((_D,), lambda i: (0,)),
            pl.BlockSpec((_D,), lambda i: (0,)),
        ],
        out_specs=pl.BlockSpec((bb, _T + 1, _D), lambda i: (i, 0, 0)),
        out_shape=jax.ShapeDtypeStruct((_B, _T + 1, _D), jnp.float32),
    )(tok_flat, td2, patient_contexts, wt, cw, ctx_token,
      fw, fb, gamma, beta)


def kernel(token_ids, time_deltas, patient_contexts, token_table, t2v_lin_w,
           t2v_lin_b, t2v_freq_w, t2v_freq_b, time_proj_w, ctx_token,
           context_proj_w, ln_gamma, ln_beta):
    idx_flat = token_ids.reshape(-1)
    tok_flat = _sc_gather(token_table, idx_flat)
    td2 = time_deltas.reshape(_B // 32, 1, 32 * _T)
    wt = time_proj_w.T
    cw = context_proj_w.T
    fw = jnp.concatenate([t2v_lin_w.reshape(1), t2v_freq_w.reshape(-1)],
                         axis=0).reshape(_T2V_DIM, 1)
    fb = jnp.concatenate([t2v_lin_b, t2v_freq_b], axis=0).reshape(_T2V_DIM, 1)
    return _dense(tok_flat, td2, patient_contexts, wt, cw, ctx_token,
                  fw, fb, ln_gamma, ln_beta)


# speedup vs baseline: 4.0860x; 1.0722x over previous
import functools
import math

import jax
import jax.numpy as jnp
from jax import lax
from jax.experimental import pallas as pl
from jax.experimental.pallas import tpu as pltpu
from jax.experimental.pallas import tpu_sc as plsc

_VOCAB = 100000
_CTX_DIM = 32
_T2V_DIM = 8
_D = 128
_B = 1024
_T = 200


def _sc_gather(table, idx_flat):
    info = plsc.get_sparse_core_info()
    nc, ns = info.num_cores, info.num_subcores
    nw = nc * ns
    n = idx_flat.shape[0]
    d = table.shape[1]
    n_per_w = n // nw
    ch = 320
    n_ch = n_per_w // ch
    mesh = plsc.VectorSubcoreMesh(core_axis_name="c", subcore_axis_name="s")

    @functools.partial(
        pl.kernel,
        mesh=mesh,
        out_type=jax.ShapeDtypeStruct((n, d), jnp.float32),
        scratch_types=[
            pltpu.VMEM((ch,), jnp.int32),
            pltpu.VMEM((ch, d), jnp.float32),
            pltpu.SemaphoreType.DMA,
        ],
    )
    def k(table_hbm, idx_hbm, out_hbm, idx_v, rows_v, sem):
        wid = lax.axis_index("s") * nc + lax.axis_index("c")
        base = wid * n_per_w

        def body(i, carry):
            off = base + i * ch
            pltpu.sync_copy(idx_hbm.at[pl.ds(off, ch)], idx_v)
            pltpu.async_copy(table_hbm.at[idx_v], rows_v, sem).wait()
            pltpu.sync_copy(rows_v, out_hbm.at[pl.ds(off, ch)])
            return carry

        lax.fori_loop(0, n_ch, body, 0)

    return k(table, idx_flat)


def _dense_body(tok_ref, td_ref, pc_ref, wt_ref, cw_ref, ctxtok_ref,
                fw_ref, fb_ref, gamma_ref, beta_ref, out_ref):
    bb = pc_ref.shape[0]
    scale_inv = 1.0 / math.sqrt(_D)
    eps = 1e-5
    gamma = gamma_ref[:].reshape(1, _D)
    beta = beta_ref[:].reshape(1, _D)

    ctx = ctxtok_ref[:].reshape(1, _D) + jnp.dot(
        pc_ref[:], cw_ref[:], preferred_element_type=jnp.float32)

    n = td_ref.shape[2]
    args8 = jnp.broadcast_to(td_ref[0], (_T2V_DIM, n)) * fw_ref[:] + fb_ref[:]
    s8 = jnp.sin(args8)
    rowmask = lax.broadcasted_iota(jnp.int32, (_T2V_DIM, n), 0) == 0
    pt = jnp.where(rowmask, args8, s8)
    time_vec = lax.dot_general(
        pt, wt_ref[:], (((0,), (0,)), ((), ())),
        preferred_element_type=jnp.float32)
    ev = (tok_ref[:] + time_vec) * scale_inv

    x = jnp.concatenate([ctx, ev], axis=0)
    jmat = jnp.full((_D, _D), 1.0 / _D, dtype=jnp.float32)
    m1 = jnp.dot(x, jmat, preferred_element_type=jnp.float32)
    m2 = jnp.dot(x * x, jmat, preferred_element_type=jnp.float32)
    var = m2 - m1 * m1
    x_n = (x - m1) * lax.rsqrt(var + eps) * gamma + beta

    out_ref[:, 0, :] = x_n[:bb, :]
    for j in range(bb):
        out_ref[j, 1:, :] = x_n[bb + j * _T:bb + (j + 1) * _T, :]


_BB = 32
_S = 4


def _dense_body_aliased(prev_ref, *refs):
    del prev_ref
    _dense_body(*refs)


def _weight_specs():
    return [
        pl.BlockSpec((_T2V_DIM, _D), lambda i: (0, 0)),
        pl.BlockSpec((_CTX_DIM, _D), lambda i: (0, 0)),
        pl.BlockSpec((_D,), lambda i: (0,)),
        pl.BlockSpec((_T2V_DIM, 1), lambda i: (0, 0)),
        pl.BlockSpec((_T2V_DIM, 1), lambda i: (0, 0)),
        pl.BlockSpec((_D,), lambda i: (0,)),
        pl.BlockSpec((_D,), lambda i: (0,)),
    ]


def _chunk_specs():
    return [
        pl.BlockSpec((_BB * _T, _D), lambda i: (i, 0)),
        pl.BlockSpec((1, 1, _BB * _T), lambda i: (i, 0, 0)),
        pl.BlockSpec((_BB, _CTX_DIM), lambda i: (i, 0)),
    ]


def _dense_chunk(out_prev, tok_s, td_s, pc_s, weights, s0, nb):
    grid = (nb // _BB,)

    def out_map(i, s0=s0):
        return (s0 + i, 0, 0)

    out_spec = pl.BlockSpec((_BB, _T + 1, _D), out_map)
    out_shape = jax.ShapeDtypeStruct((_B, _T + 1, _D), jnp.float32)
    if out_prev is None:
        return pl.pallas_call(
            _dense_body,
            grid=grid,
            in_specs=_chunk_specs() + _weight_specs(),
            out_specs=out_spec,
            out_shape=out_shape,
        )(tok_s, td_s, pc_s, *weights)
    return pl.pallas_call(
        _dense_body_aliased,
        grid=grid,
        in_specs=[pl.BlockSpec(memory_space=pl.ANY)]
        + _chunk_specs() + _weight_specs(),
        out_specs=out_spec,
        out_shape=out_shape,
        input_output_aliases={0: 0},
    )(out_prev, tok_s, td_s, pc_s, *weights)


def kernel(token_ids, time_deltas, patient_contexts, token_table, t2v_lin_w,
           t2v_lin_b, t2v_freq_w, t2v_freq_b, time_proj_w, ctx_token,
           context_proj_w, ln_gamma, ln_beta):
    idx_flat = token_ids.reshape(-1)
    wt = time_proj_w.T
    cw = context_proj_w.T
    fw = jnp.concatenate([t2v_lin_w.reshape(1), t2v_freq_w.reshape(-1)],
                         axis=0).reshape(_T2V_DIM, 1)
    fb = jnp.concatenate([t2v_lin_b, t2v_freq_b], axis=0).reshape(_T2V_DIM, 1)
    weights = (wt, cw, ctx_token, fw, fb, ln_gamma, ln_beta)

    nb = _B // _S
    nr = nb * _T
    td3 = time_deltas.reshape(_B // _BB, 1, _BB * _T)
    toks = [_sc_gather(token_table, idx_flat[s * nr:(s + 1) * nr])
            for s in range(_S)]
    out = None
    for s in range(_S):
        out = _dense_chunk(
            out, toks[s],
            td3[s * (nb // _BB):(s + 1) * (nb // _BB)],
            patient_contexts[s * nb:(s + 1) * nb],
            weights, s0=s * (nb // _BB), nb=nb)
    return out


# speedup vs baseline: 4.1992x; 1.0277x over previous
import functools
import math

import jax
import jax.numpy as jnp
from jax import lax
from jax.experimental import pallas as pl
from jax.experimental.pallas import tpu as pltpu
from jax.experimental.pallas import tpu_sc as plsc

_VOCAB = 100000
_CTX_DIM = 32
_T2V_DIM = 8
_D = 128
_B = 1024
_T = 200


def _sc_gather(table, idx_flat):
    info = plsc.get_sparse_core_info()
    nc, ns = info.num_cores, info.num_subcores
    nw = nc * ns
    n = idx_flat.shape[0]
    d = table.shape[1]
    n_per_w = n // nw
    ch = 200
    n_ch = n_per_w // ch
    mesh = plsc.VectorSubcoreMesh(core_axis_name="c", subcore_axis_name="s")

    @functools.partial(
        pl.kernel,
        mesh=mesh,
        out_type=jax.ShapeDtypeStruct((n, d), jnp.float32),
        scratch_types=[
            pltpu.VMEM((ch,), jnp.int32),
            pltpu.VMEM((ch,), jnp.int32),
            pltpu.VMEM((ch, d), jnp.float32),
            pltpu.VMEM((ch, d), jnp.float32),
            pltpu.SemaphoreType.DMA,
            pltpu.SemaphoreType.DMA,
        ],
    )
    def k(table_hbm, idx_hbm, out_hbm, idx_v0, idx_v1, rows_v0, rows_v1,
          sem0, sem1):
        wid = lax.axis_index("s") * nc + lax.axis_index("c")
        base = wid * n_per_w
        idx_bufs = (idx_v0, idx_v1)
        row_bufs = (rows_v0, rows_v1)
        sem_bufs = (sem0, sem1)
        pltpu.sync_copy(idx_hbm.at[pl.ds(base, ch)], idx_bufs[0])
        copies = [pltpu.async_copy(
            table_hbm.at[idx_bufs[0]], row_bufs[0], sem_bufs[0])]
        for g in range(n_ch):
            b = g % 2
            if g + 1 < n_ch:
                nxt = 1 - b
                off = base + (g + 1) * ch
                pltpu.sync_copy(idx_hbm.at[pl.ds(off, ch)], idx_bufs[nxt])
                copies.append(pltpu.async_copy(
                    table_hbm.at[idx_bufs[nxt]], row_bufs[nxt], sem_bufs[nxt]))
            copies[g].wait()
            pltpu.sync_copy(row_bufs[b],
                            out_hbm.at[pl.ds(base + g * ch, ch)])

    return k(table, idx_flat)


def _dense_body(tok_ref, td_ref, pc_ref, wt_ref, cw_ref, ctxtok_ref,
                fw_ref, fb_ref, gamma_ref, beta_ref, out_ref):
    bb = pc_ref.shape[0]
    scale_inv = 1.0 / math.sqrt(_D)
    eps = 1e-5
    gamma = gamma_ref[:].reshape(1, _D)
    beta = beta_ref[:].reshape(1, _D)

    ctx = ctxtok_ref[:].reshape(1, _D) + jnp.dot(
        pc_ref[:], cw_ref[:], preferred_element_type=jnp.float32)

    n = td_ref.shape[2]
    args8 = jnp.broadcast_to(td_ref[0], (_T2V_DIM, n)) * fw_ref[:] + fb_ref[:]
    s8 = jnp.sin(args8)
    rowmask = lax.broadcasted_iota(jnp.int32, (_T2V_DIM, n), 0) == 0
    pt = jnp.where(rowmask, args8, s8)
    time_vec = lax.dot_general(
        pt, wt_ref[:], (((0,), (0,)), ((), ())),
        preferred_element_type=jnp.float32)
    ev = (tok_ref[:] + time_vec) * scale_inv

    x = jnp.concatenate([ctx, ev], axis=0)
    jmat = jnp.full((_D, _D), 1.0 / _D, dtype=jnp.float32)
    m1 = jnp.dot(x, jmat, preferred_element_type=jnp.float32)
    m2 = jnp.dot(x * x, jmat, preferred_element_type=jnp.float32)
    var = m2 - m1 * m1
    x_n = (x - m1) * lax.rsqrt(var + eps) * gamma + beta

    out_ref[:, 0, :] = x_n[:bb, :]
    for j in range(bb):
        out_ref[j, 1:, :] = x_n[bb + j * _T:bb + (j + 1) * _T, :]


_BB = 32
_S = 4


def _dense_body_aliased(prev_ref, *refs):
    del prev_ref
    _dense_body(*refs)


def _weight_specs():
    return [
        pl.BlockSpec((_T2V_DIM, _D), lambda i: (0, 0)),
        pl.BlockSpec((_CTX_DIM, _D), lambda i: (0, 0)),
        pl.BlockSpec((_D,), lambda i: (0,)),
        pl.BlockSpec((_T2V_DIM, 1), lambda i: (0, 0)),
        pl.BlockSpec((_T2V_DIM, 1), lambda i: (0, 0)),
        pl.BlockSpec((_D,), lambda i: (0,)),
        pl.BlockSpec((_D,), lambda i: (0,)),
    ]


def _chunk_specs():
    return [
        pl.BlockSpec((_BB * _T, _D), lambda i: (i, 0)),
        pl.BlockSpec((1, 1, _BB * _T), lambda i: (i, 0, 0)),
        pl.BlockSpec((_BB, _CTX_DIM), lambda i: (i, 0)),
    ]


def _dense_chunk(out_prev, tok_s, td_s, pc_s, weights, s0, nb):
    grid = (nb // _BB,)

    def out_map(i, s0=s0):
        return (s0 + i, 0, 0)

    out_spec = pl.BlockSpec((_BB, _T + 1, _D), out_map)
    out_shape = jax.ShapeDtypeStruct((_B, _T + 1, _D), jnp.float32)
    if out_prev is None:
        return pl.pallas_call(
            _dense_body,
            grid=grid,
            in_specs=_chunk_specs() + _weight_specs(),
            out_specs=out_spec,
            out_shape=out_shape,
        )(tok_s, td_s, pc_s, *weights)
    return pl.pallas_call(
        _dense_body_aliased,
        grid=grid,
        in_specs=[pl.BlockSpec(memory_space=pl.ANY)]
        + _chunk_specs() + _weight_specs(),
        out_specs=out_spec,
        out_shape=out_shape,
        input_output_aliases={0: 0},
    )(out_prev, tok_s, td_s, pc_s, *weights)


def kernel(token_ids, time_deltas, patient_contexts, token_table, t2v_lin_w,
           t2v_lin_b, t2v_freq_w, t2v_freq_b, time_proj_w, ctx_token,
           context_proj_w, ln_gamma, ln_beta):
    idx_flat = token_ids.reshape(-1)
    wt = time_proj_w.T
    cw = context_proj_w.T
    fw = jnp.concatenate([t2v_lin_w.reshape(1), t2v_freq_w.reshape(-1)],
                         axis=0).reshape(_T2V_DIM, 1)
    fb = jnp.concatenate([t2v_lin_b, t2v_freq_b], axis=0).reshape(_T2V_DIM, 1)
    weights = (wt, cw, ctx_token, fw, fb, ln_gamma, ln_beta)

    sizes = (32, 320, 320, 352)
    td3 = time_deltas.reshape(_B // _BB, 1, _BB * _T)
    starts = [sum(sizes[:s]) for s in range(len(sizes))]
    toks = [_sc_gather(token_table, idx_flat[b0 * _T:(b0 + nb) * _T])
            for b0, nb in zip(starts, sizes)]
    out = None
    for s, (b0, nb) in enumerate(zip(starts, sizes)):
        out = _dense_chunk(
            out, toks[s],
            td3[b0 // _BB:(b0 + nb) // _BB],
            patient_contexts[b0:b0 + nb],
            weights, s0=b0 // _BB, nb=nb)
    return out


# speedup vs baseline: 5.3157x; 1.2659x over previous
import functools
import math

import jax
import jax.numpy as jnp
from jax import lax
from jax.experimental import pallas as pl
from jax.experimental.pallas import tpu as pltpu
from jax.experimental.pallas import tpu_sc as plsc

_VOCAB = 100000
_CTX_DIM = 32
_T2V_DIM = 8
_D = 128
_B = 1024
_T = 200

_TCH = (2, 63, 66, 69)
_R = 3 * _B


def _pick_chunk(n_per_w):
    for c in range(320, 0, -8):
        if n_per_w % c == 0:
            return c
    return n_per_w


def _sc_gather(table, idx_flat):
    info = plsc.get_sparse_core_info()
    nc, ns = info.num_cores, info.num_subcores
    nw = nc * ns
    n = idx_flat.shape[0]
    d = table.shape[1]
    n_per_w = n // nw
    ch = _pick_chunk(n_per_w)
    n_ch = n_per_w // ch
    mesh = plsc.VectorSubcoreMesh(core_axis_name="c", subcore_axis_name="s")

    @functools.partial(
        pl.kernel,
        mesh=mesh,
        out_type=jax.ShapeDtypeStruct((n, d), jnp.float32),
        scratch_types=[
            pltpu.VMEM((ch,), jnp.int32),
            pltpu.VMEM((ch,), jnp.int32),
            pltpu.VMEM((ch, d), jnp.float32),
            pltpu.VMEM((ch, d), jnp.float32),
            pltpu.SemaphoreType.DMA,
            pltpu.SemaphoreType.DMA,
        ],
    )
    def k(table_hbm, idx_hbm, out_hbm, idx_v0, idx_v1, rows_v0, rows_v1,
          sem0, sem1):
        wid = lax.axis_index("s") * nc + lax.axis_index("c")
        base = wid * n_per_w
        idx_bufs = (idx_v0, idx_v1)
        row_bufs = (rows_v0, rows_v1)
        sem_bufs = (sem0, sem1)
        pltpu.sync_copy(idx_hbm.at[pl.ds(base, ch)], idx_bufs[0])
        copies = [pltpu.async_copy(
            table_hbm.at[idx_bufs[0]], row_bufs[0], sem_bufs[0])]
        for g in range(n_ch):
            b = g % 2
            if g + 1 < n_ch:
                nxt = 1 - b
                off = base + (g + 1) * ch
                pltpu.sync_copy(idx_hbm.at[pl.ds(off, ch)], idx_bufs[nxt])
                copies.append(pltpu.async_copy(
                    table_hbm.at[idx_bufs[nxt]], row_bufs[nxt], sem_bufs[nxt]))
            copies[g].wait()
            pltpu.sync_copy(row_bufs[b],
                            out_hbm.at[pl.ds(base + g * ch, ch)])

    return k(table, idx_flat)


def _layernorm(x, gamma, beta):
    jmat = jnp.full((_D, _D), 1.0 / _D, dtype=jnp.float32)
    m1 = jnp.dot(x, jmat, preferred_element_type=jnp.float32)
    m2 = jnp.dot(x * x, jmat, preferred_element_type=jnp.float32)
    var = m2 - m1 * m1
    return (x - m1) * lax.rsqrt(var + 1e-5) * gamma + beta


def _time_vec(td_ref, fw_ref, fb_ref, wt_ref):
    n = td_ref.shape[2]
    args8 = jnp.broadcast_to(td_ref[0], (_T2V_DIM, n)) * fw_ref[:] + fb_ref[:]
    s8 = jnp.sin(args8)
    rowmask = lax.broadcasted_iota(jnp.int32, (_T2V_DIM, n), 0) == 0
    pt = jnp.where(rowmask, args8, s8)
    return lax.dot_general(
        pt, wt_ref[:], (((0,), (0,)), ((), ())),
        preferred_element_type=jnp.float32)


def _ev_body(tok_ref, td_ref, wt_ref, fw_ref, fb_ref, gamma_ref, beta_ref,
             out_ref):
    gamma = gamma_ref[:].reshape(1, _D)
    beta = beta_ref[:].reshape(1, _D)
    tv = _time_vec(td_ref, fw_ref, fb_ref, wt_ref)
    ev = (tok_ref[:] + tv) * (1.0 / math.sqrt(_D))
    out_ref[:] = _layernorm(ev, gamma, beta)


def _head_body(tok_ref, td_ref, pc_ref, wt_ref, cw_ref, ctxtok_ref,
               fw_ref, fb_ref, gamma_ref, beta_ref, out_ref):
    i = pl.program_id(0)
    gamma = gamma_ref[:].reshape(1, _D)
    beta = beta_ref[:].reshape(1, _D)
    tv = _time_vec(td_ref, fw_ref, fb_ref, wt_ref)
    ev = (tok_ref[:] + tv) * (1.0 / math.sqrt(_D))
    ctx = ctxtok_ref[:].reshape(1, _D) + jnp.dot(
        pc_ref[:], cw_ref[:], preferred_element_type=jnp.float32)
    x = jnp.where(i == 0, ctx, ev)
    out_ref[:] = _layernorm(x, gamma, beta)


_NROW = (_T + 1) * _B


def _small_specs():
    return [
        pl.BlockSpec((_T2V_DIM, _D), lambda i: (0, 0)),
        pl.BlockSpec((_T2V_DIM, 1), lambda i: (0, 0)),
        pl.BlockSpec((_T2V_DIM, 1), lambda i: (0, 0)),
        pl.BlockSpec((_D,), lambda i: (0,)),
        pl.BlockSpec((_D,), lambda i: (0,)),
    ]


def _head_chunk(tok, td, pc, wt, cw, ctok, fw, fb, gamma, beta, nsteps):
    grid = (nsteps,)
    in_specs = [
        pl.BlockSpec((_B, _D), lambda i: (jnp.maximum(i - 1, 0), 0)),
        pl.BlockSpec((1, 1, _B), lambda i: (i, 0, 0)),
        pl.BlockSpec((_B, _CTX_DIM), lambda i: (0, 0)),
        pl.BlockSpec((_T2V_DIM, _D), lambda i: (0, 0)),
        pl.BlockSpec((_CTX_DIM, _D), lambda i: (0, 0)),
        pl.BlockSpec((_D,), lambda i: (0,)),
        pl.BlockSpec((_T2V_DIM, 1), lambda i: (0, 0)),
        pl.BlockSpec((_T2V_DIM, 1), lambda i: (0, 0)),
        pl.BlockSpec((_D,), lambda i: (0,)),
        pl.BlockSpec((_D,), lambda i: (0,)),
    ]
    return pl.pallas_call(
        _head_body,
        grid=grid,
        in_specs=in_specs,
        out_specs=pl.BlockSpec((_B, _D), lambda i: (i, 0)),
        out_shape=jax.ShapeDtypeStruct((_NROW, _D), jnp.float32),
    )(tok, td, pc, wt, cw, ctok, fw, fb, gamma, beta)


def _ev_chunk(out_prev, tok, td, weights, r0, nsteps):
    def out_map(i, r0=r0):
        return (r0 + i, 0)

    return pl.pallas_call(
        lambda prev_ref, *refs: _ev_body(*refs),
        grid=(nsteps,),
        in_specs=[
            pl.BlockSpec(memory_space=pl.ANY),
            pl.BlockSpec((_R, _D), lambda i: (i, 0)),
            pl.BlockSpec((1, 1, _R), lambda i: (i, 0, 0)),
        ] + _small_specs(),
        out_specs=pl.BlockSpec((_R, _D), out_map),
        out_shape=jax.ShapeDtypeStruct((_NROW, _D), jnp.float32),
        input_output_aliases={0: 0},
    )(out_prev, tok, td, *weights)


def kernel(token_ids, time_deltas, patient_contexts, token_table, t2v_lin_w,
           t2v_lin_b, t2v_freq_w, t2v_freq_b, time_proj_w, ctx_token,
           context_proj_w, ln_gamma, ln_beta):
    idx_tm = token_ids.T.reshape(-1)
    td_tm = time_deltas.T.reshape(-1)

    wt = time_proj_w.T
    cw = context_proj_w.T
    fw = jnp.concatenate([t2v_lin_w.reshape(1), t2v_freq_w.reshape(-1)],
                         axis=0).reshape(_T2V_DIM, 1)
    fb = jnp.concatenate([t2v_lin_b, t2v_freq_b], axis=0).reshape(_T2V_DIM, 1)
    weights = (wt, fw, fb, ln_gamma, ln_beta)

    t0s = [sum(_TCH[:s]) for s in range(len(_TCH))]
    toks = [_sc_gather(token_table, idx_tm[t0 * _B:(t0 + tc) * _B])
            for t0, tc in zip(t0s, _TCH)]

    td_head = jnp.concatenate([jnp.zeros((_B,), jnp.float32),
                               td_tm[:_TCH[0] * _B]]).reshape(-1, 1, _B)
    out = _head_chunk(toks[0], td_head, patient_contexts, wt, cw, ctx_token,
                      fw, fb, ln_gamma, ln_beta, nsteps=1 + _TCH[0])

    for s in range(1, len(_TCH)):
        t0, tc = t0s[s], _TCH[s]
        td_s = td_tm[t0 * _B:(t0 + tc) * _B].reshape(-1, 1, _R)
        out = _ev_chunk(out, toks[s], td_s, weights,
                        r0=(1 + t0) * _B // _R, nsteps=tc * _B // _R)

    return out.reshape(_T + 1, _B, _D).swapaxes(0, 1)
